# baseline (speedup 1.0000x reference)
import jax
import jax.numpy as jnp
from jax.experimental import pallas as pl
from jax.experimental.pallas import tpu as pltpu

_B, _D = 65536, 512
_BLK = 1024
_NJ = _B // (2 * _BLK)


def _body(fs_ref, ft_ref, out_ref):
    j = pl.program_id(1)

    @pl.when(j == 0)
    def _init():
        out_ref[...] = jnp.zeros_like(out_ref)

    prod = fs_ref[...] * ft_ref[...]
    out_ref[...] += jnp.sum(prod.reshape(_BLK // 8, 8, _D), axis=0)


def kernel(f_s, f_t):
    in_spec = pl.BlockSpec((_BLK, _D), lambda i, j: (i * _NJ + j, 0))
    partials = pl.pallas_call(
        _body,
        grid=(2, _NJ),
        in_specs=[in_spec, in_spec],
        out_specs=pl.BlockSpec((None, 8, _D), lambda i, j: (i, 0, 0)),
        out_shape=jax.ShapeDtypeStruct((2, 8, _D), jnp.float32),
        compiler_params=pltpu.CompilerParams(
            dimension_semantics=("parallel", "arbitrary"),
        ),
    )(f_s, f_t)
    return -jnp.sum(partials, keepdims=True).reshape(1) / _B


# speedup vs baseline: 1.1257x; 1.1257x over previous
import jax
import jax.numpy as jnp
from jax.experimental import pallas as pl
from jax.experimental.pallas import tpu as pltpu

_B, _D = 65536, 512
_BLK = 2048
_NJ = _B // _BLK


def _body(fs_ref, ft_ref, out_ref, acc_ref):
    j = pl.program_id(0)

    @pl.when(j == 0)
    def _init():
        acc_ref[...] = jnp.zeros_like(acc_ref)

    prod = fs_ref[...] * ft_ref[...]
    acc_ref[...] += jnp.sum(prod.reshape(_BLK // 8, 8, _D), axis=0)

    @pl.when(j == _NJ - 1)
    def _fini():
        cols = jnp.sum(acc_ref[...], axis=0, keepdims=True)
        out_ref[...] = -jnp.sum(cols, axis=1, keepdims=True) / _B


def kernel(f_s, f_t):
    in_spec = pl.BlockSpec((_BLK, _D), lambda j: (j, 0))
    loss = pl.pallas_call(
        _body,
        grid=(_NJ,),
        in_specs=[in_spec, in_spec],
        out_specs=pl.BlockSpec((1, 1), lambda j: (0, 0)),
        out_shape=jax.ShapeDtypeStruct((1, 1), jnp.float32),
        scratch_shapes=[pltpu.VMEM((8, _D), jnp.float32)],
        compiler_params=pltpu.CompilerParams(
            dimension_semantics=("arbitrary",),
        ),
    )(f_s, f_t)
    return loss.reshape(1)
